# baseline (speedup 1.0000x reference)
import jax
import jax.numpy as jnp
import numpy as np
from jax import lax
from jax.experimental import pallas as pl
from jax.experimental.pallas import tpu as pltpu
from jax.experimental.pallas import tpu_sc as plsc

N_NODES = 10000
N_EDGES = 320000
D = 128
NC = 2
NS = 16
NT = NC * NS
EPT = N_EDGES // NT
KH = 40
SUBS = EPT // KH
PAIRS = SUBS // 2
NROWS = 10240
RPT = NROWS // NS


def _scale_half(rows_bf, rows_f, lane_ranges):
    for wvec, lanes, row0 in lane_ranges:
        for j in lanes:
            wb = jnp.full((16,), wvec[j], jnp.float32)
            r = row0 + (j - lanes[0])
            for q in range(D // 32):
                v = rows_bf[r, pl.ds(32 * q, 32)]
                a, b = plsc.unpack(v, format=plsc.PackFormat.INTERLEAVED)
                rows_f[r, pl.ds(32 * q, 16)] = a * wb
                rows_f[r, pl.ds(32 * q + 16, 16)] = b * wb


def _sc_body(h_hbm, src_hbm, dst_hbm, w_hbm, msg_hbm, deg_hbm,
             src_v, dst_v, w_v, rows_a, rows_b, rows_f0,
             ones_v, zdeg_v, acc_sh, dacc_sh, sem_a, sem_b, sem_s0):
    cid = lax.axis_index("c")
    sid = lax.axis_index("s")
    tile = cid * NS + sid

    zeros16 = jnp.zeros((16,), jnp.float32)
    ones16 = jnp.ones((16,), jnp.float32)

    def _zero_row(r, _):
        for f in range(D // 16):
            rows_f0[r, pl.ds(16 * f, 16)] = zeros16
        return _
    lax.fori_loop(0, 2 * KH, _zero_row, None)
    for j in range(RPT // (2 * KH)):
        pltpu.sync_copy(rows_f0,
                        acc_sh.at[pl.ds(sid * RPT + j * 2 * KH, 2 * KH)])
    for g in range(RPT // 16):
        zdeg_v[pl.ds(16 * g, 16)] = zeros16
    pltpu.sync_copy(zdeg_v, dacc_sh.at[pl.ds(sid * RPT, RPT)])
    for g in range(2 * KH // 16):
        ones_v[pl.ds(16 * g, 16)] = ones16

    pltpu.sync_copy(src_hbm.at[tile], src_v)
    pltpu.sync_copy(dst_hbm.at[tile], dst_v)
    pltpu.sync_copy(w_hbm.at[tile], w_v)

    plsc.subcore_barrier()

    pltpu.async_copy(h_hbm.at[src_v.at[0]], rows_a, sem_a)
    pltpu.async_copy(h_hbm.at[src_v.at[1]], rows_b, sem_b)

    def _do_pair(p, _):
        s0 = 2 * p
        s1 = 2 * p + 1

        pltpu.make_async_copy(h_hbm.at[pl.ds(0, KH)], rows_a, sem_a).wait()

        @pl.when(p >= 1)
        def _():
            pltpu.make_async_copy(rows_f0, acc_sh.at[dst_v.at[0]],
                                  sem_s0).wait()
            pltpu.make_async_copy(ones_v, dacc_sh.at[dst_v.at[0]],
                                  sem_s0).wait()

        w0 = w_v[p, pl.ds(0, 16)]
        w1 = w_v[p, pl.ds(16, 16)]
        w2 = w_v[p, pl.ds(32, 16)]
        _scale_half(rows_a, rows_f0, [
            (w0, range(0, 16), 0),
            (w1, range(0, 16), 16),
            (w2, range(0, 8), 32),
        ])

        @pl.when(s0 + 2 < SUBS)
        def _():
            pltpu.async_copy(h_hbm.at[src_v.at[s0 + 2]], rows_a, sem_a)

        pltpu.make_async_copy(h_hbm.at[pl.ds(0, KH)], rows_b, sem_b).wait()
        w2b = w_v[p, pl.ds(32, 16)]
        w3 = w_v[p, pl.ds(48, 16)]
        w4 = w_v[p, pl.ds(64, 16)]
        _scale_half(rows_b, rows_f0, [
            (w2b, range(8, 16), 40),
            (w3, range(0, 16), 48),
            (w4, range(0, 16), 64),
        ])

        @pl.when(s1 + 2 < SUBS)
        def _():
            pltpu.async_copy(h_hbm.at[src_v.at[s1 + 2]], rows_b, sem_b)

        pltpu.async_copy(rows_f0, acc_sh.at[dst_v.at[p]], sem_s0, add=True)
        pltpu.async_copy(ones_v, dacc_sh.at[dst_v.at[p]], sem_s0, add=True)
        return _
    lax.fori_loop(0, PAIRS, _do_pair, None)

    pltpu.make_async_copy(rows_f0, acc_sh.at[dst_v.at[0]], sem_s0).wait()
    pltpu.make_async_copy(ones_v, dacc_sh.at[dst_v.at[0]], sem_s0).wait()

    plsc.subcore_barrier()
    pltpu.sync_copy(acc_sh.at[pl.ds(sid * RPT, RPT)],
                    msg_hbm.at[cid, pl.ds(sid * RPT, RPT)])
    pltpu.sync_copy(dacc_sh.at[pl.ds(sid * RPT, RPT)],
                    deg_hbm.at[cid, pl.ds(sid * RPT, RPT)])


def _make_sc():
    mesh = plsc.VectorSubcoreMesh(core_axis_name="c", subcore_axis_name="s")
    return pl.kernel(
        _sc_body,
        out_type=(
            jax.ShapeDtypeStruct((NC, NROWS, D), jnp.float32),
            jax.ShapeDtypeStruct((NC, NROWS), jnp.float32),
        ),
        mesh=mesh,
        compiler_params=pltpu.CompilerParams(
            use_tc_tiling_on_sc=False, needs_layout_passes=False),
        scratch_types=[
            pltpu.VMEM((SUBS, KH), jnp.int32),
            pltpu.VMEM((PAIRS, 2 * KH), jnp.int32),
            pltpu.VMEM((PAIRS, 2 * KH), jnp.float32),
            pltpu.VMEM((KH, D), jnp.bfloat16),
            pltpu.VMEM((KH, D), jnp.bfloat16),
            pltpu.VMEM((2 * KH, D), jnp.float32),
            pltpu.VMEM((2 * KH,), jnp.float32),
            pltpu.VMEM((RPT,), jnp.float32),
            pltpu.VMEM_SHARED((NROWS, D), jnp.float32),
            pltpu.VMEM_SHARED((NROWS,), jnp.float32),
            pltpu.SemaphoreType.DMA,
            pltpu.SemaphoreType.DMA,
            pltpu.SemaphoreType.DMA,
        ],
    )


def _tc_body(h_ref, p0_ref, p1_ref, d0_ref, d1_ref, w1_ref, w2_ref, b_ref,
             o_ref):
    msg = p0_ref[0] + p1_ref[0]
    deg = d0_ref[...] + d1_ref[...]
    h_n = msg / jnp.maximum(deg, 1.0)
    o_ref[...] = (
        jnp.dot(h_ref[...], w1_ref[...], preferred_element_type=jnp.float32)
        + jnp.dot(h_n, w2_ref[...], preferred_element_type=jnp.float32)
        + b_ref[...]
    )


def _tc_finish(h, msg, d0, d1, w1t, w2t, b2):
    R = 1000
    grid = (N_NODES // R,)
    return pl.pallas_call(
        _tc_body,
        grid=grid,
        in_specs=[
            pl.BlockSpec((R, D), lambda i: (i, 0)),
            pl.BlockSpec((1, R, D), lambda i: (0, i, 0)),
            pl.BlockSpec((1, R, D), lambda i: (1, i, 0)),
            pl.BlockSpec((R, 1), lambda i: (i, 0)),
            pl.BlockSpec((R, 1), lambda i: (i, 0)),
            pl.BlockSpec((D, D), lambda i: (0, 0)),
            pl.BlockSpec((D, D), lambda i: (0, 0)),
            pl.BlockSpec((1, D), lambda i: (0, 0)),
        ],
        out_specs=pl.BlockSpec((R, D), lambda i: (i, 0)),
        out_shape=jax.ShapeDtypeStruct((N_NODES, D), jnp.float32),
    )(h, msg, msg, d0, d1, w1t, w2t, b2)


_PERM = np.asarray(
    [32 * q + o
     for q in range(D // 32)
     for o in [2 * k for k in range(16)] + [2 * k + 1 for k in range(16)]],
    dtype=np.int32)


def kernel(h, edge_index, w, W, b):
    src2 = edge_index[0].reshape(NT, SUBS, KH)
    dst2 = edge_index[1].reshape(NT, PAIRS, 2 * KH)
    w2 = w.reshape(NT, PAIRS, 2 * KH)
    h_bf = h.astype(jnp.bfloat16)
    msg, deg = _make_sc()(h_bf, src2, dst2, w2)
    w1t = W[:, :D].T
    w2t = W[:, D:].T[_PERM]
    b2 = b.reshape(1, D)
    return _tc_finish(h, msg,
                      deg[0, :N_NODES, None], deg[1, :N_NODES, None],
                      w1t, w2t, b2)


# speedup vs baseline: 1.0120x; 1.0120x over previous
import jax
import jax.numpy as jnp
import numpy as np
from jax import lax
from jax.experimental import pallas as pl
from jax.experimental.pallas import tpu as pltpu
from jax.experimental.pallas import tpu_sc as plsc

N_NODES = 10000
N_EDGES = 320000
D = 128
NC = 2
NS = 16
NT = NC * NS
EPT = N_EDGES // NT
KH = 40
SUBS = EPT // KH
PAIRS = SUBS // 2
NROWS = 10240
RPT = NROWS // NS


def _scale_half(rows_bf, rows_f, lane_ranges):
    for wvec, lanes, row0 in lane_ranges:
        for j in lanes:
            wb = jnp.full((16,), wvec[j], jnp.float32)
            r = row0 + (j - lanes[0])
            for q in range(D // 32):
                v = rows_bf[r, pl.ds(32 * q, 32)]
                a, b = plsc.unpack(v, format=plsc.PackFormat.INTERLEAVED)
                rows_f[r, pl.ds(32 * q, 16)] = a * wb
                rows_f[r, pl.ds(32 * q + 16, 16)] = b * wb


def _sc_body(h_hbm, src_hbm, dst_hbm, w_hbm, msg_hbm, deg_hbm,
             src_v, dst_v, w_v, rows_a, rows_b, rows_fa, rows_fb,
             ones_v, zdeg_v, acc_sh, dacc_sh, sem_a, sem_b, sem_sa, sem_sb):
    cid = lax.axis_index("c")
    sid = lax.axis_index("s")
    tile = cid * NS + sid

    zeros16 = jnp.zeros((16,), jnp.float32)
    ones16 = jnp.ones((16,), jnp.float32)

    def _zero_row(r, _):
        for f in range(D // 16):
            rows_fa[r, pl.ds(16 * f, 16)] = zeros16
        return _
    lax.fori_loop(0, KH, _zero_row, None)
    for j in range(RPT // KH):
        pltpu.sync_copy(rows_fa, acc_sh.at[pl.ds(sid * RPT + j * KH, KH)])
    for g in range(RPT // 16):
        zdeg_v[pl.ds(16 * g, 16)] = zeros16
    pltpu.sync_copy(zdeg_v, dacc_sh.at[pl.ds(sid * RPT, RPT)])
    for g in range(KH // 16):
        ones_v[pl.ds(16 * g, 16)] = ones16
    ones_v[pl.ds(KH - 16, 16)] = ones16

    pltpu.sync_copy(src_hbm.at[tile], src_v)
    pltpu.sync_copy(dst_hbm.at[tile], dst_v)
    pltpu.sync_copy(w_hbm.at[tile], w_v)

    plsc.subcore_barrier()

    pltpu.async_copy(h_hbm.at[src_v.at[0]], rows_a, sem_a)
    pltpu.async_copy(h_hbm.at[src_v.at[1]], rows_b, sem_b)

    def _pair(i, _):
        s0 = 2 * i
        s1 = 2 * i + 1

        pltpu.make_async_copy(h_hbm.at[pl.ds(0, KH)], rows_a, sem_a).wait()

        @pl.when(i > 0)
        def _():
            pltpu.make_async_copy(rows_fa, acc_sh.at[dst_v.at[0]],
                                  sem_sa).wait()
            pltpu.make_async_copy(ones_v, dacc_sh.at[dst_v.at[0]],
                                  sem_sa).wait()

        w0 = w_v[i, pl.ds(0, 16)]
        w1 = w_v[i, pl.ds(16, 16)]
        w2 = w_v[i, pl.ds(32, 16)]
        _scale_half(rows_a, rows_fa, [
            (w0, range(0, 16), 0),
            (w1, range(0, 16), 16),
            (w2, range(0, 8), 32),
        ])

        @pl.when(s0 + 2 < SUBS)
        def _():
            pltpu.async_copy(h_hbm.at[src_v.at[s0 + 2]], rows_a, sem_a)

        pltpu.async_copy(rows_fa, acc_sh.at[dst_v.at[s0]], sem_sa, add=True)
        pltpu.async_copy(ones_v, dacc_sh.at[dst_v.at[s0]], sem_sa, add=True)

        pltpu.make_async_copy(h_hbm.at[pl.ds(0, KH)], rows_b, sem_b).wait()

        @pl.when(i > 0)
        def _():
            pltpu.make_async_copy(rows_fb, acc_sh.at[dst_v.at[0]],
                                  sem_sb).wait()
            pltpu.make_async_copy(ones_v, dacc_sh.at[dst_v.at[0]],
                                  sem_sb).wait()

        w2b = w_v[i, pl.ds(32, 16)]
        w3 = w_v[i, pl.ds(48, 16)]
        w4 = w_v[i, pl.ds(64, 16)]
        _scale_half(rows_b, rows_fb, [
            (w2b, range(8, 16), 0),
            (w3, range(0, 16), 8),
            (w4, range(0, 16), 24),
        ])

        @pl.when(s1 + 2 < SUBS)
        def _():
            pltpu.async_copy(h_hbm.at[src_v.at[s1 + 2]], rows_b, sem_b)

        pltpu.async_copy(rows_fb, acc_sh.at[dst_v.at[s1]], sem_sb, add=True)
        pltpu.async_copy(ones_v, dacc_sh.at[dst_v.at[s1]], sem_sb, add=True)

        return _
    lax.fori_loop(0, PAIRS, _pair, None)

    pltpu.make_async_copy(rows_fa, acc_sh.at[dst_v.at[0]], sem_sa).wait()
    pltpu.make_async_copy(ones_v, dacc_sh.at[dst_v.at[0]], sem_sa).wait()
    pltpu.make_async_copy(rows_fb, acc_sh.at[dst_v.at[0]], sem_sb).wait()
    pltpu.make_async_copy(ones_v, dacc_sh.at[dst_v.at[0]], sem_sb).wait()

    plsc.subcore_barrier()
    pltpu.sync_copy(acc_sh.at[pl.ds(sid * RPT, RPT)],
                    msg_hbm.at[cid, pl.ds(sid * RPT, RPT)])
    pltpu.sync_copy(dacc_sh.at[pl.ds(sid * RPT, RPT)],
                    deg_hbm.at[cid, pl.ds(sid * RPT, RPT)])


def _make_sc():
    mesh = plsc.VectorSubcoreMesh(core_axis_name="c", subcore_axis_name="s")
    return pl.kernel(
        _sc_body,
        out_type=(
            jax.ShapeDtypeStruct((NC, NROWS, D), jnp.float32),
            jax.ShapeDtypeStruct((NC, NROWS), jnp.float32),
        ),
        mesh=mesh,
        compiler_params=pltpu.CompilerParams(
            use_tc_tiling_on_sc=False, needs_layout_passes=False),
        scratch_types=[
            pltpu.VMEM((SUBS, KH), jnp.int32),
            pltpu.VMEM((SUBS, KH), jnp.int32),
            pltpu.VMEM((PAIRS, 2 * KH), jnp.float32),
            pltpu.VMEM((KH, D), jnp.bfloat16),
            pltpu.VMEM((KH, D), jnp.bfloat16),
            pltpu.VMEM((KH, D), jnp.float32),
            pltpu.VMEM((KH, D), jnp.float32),
            pltpu.VMEM((KH,), jnp.float32),
            pltpu.VMEM((RPT,), jnp.float32),
            pltpu.VMEM_SHARED((NROWS, D), jnp.float32),
            pltpu.VMEM_SHARED((NROWS,), jnp.float32),
            pltpu.SemaphoreType.DMA,
            pltpu.SemaphoreType.DMA,
            pltpu.SemaphoreType.DMA,
            pltpu.SemaphoreType.DMA,
        ],
    )


def _tc_body(h_ref, p0_ref, p1_ref, d0_ref, d1_ref, w1_ref, w2_ref, b_ref,
             o_ref):
    msg = p0_ref[0] + p1_ref[0]
    deg = d0_ref[...] + d1_ref[...]
    h_n = msg / jnp.maximum(deg, 1.0)
    o_ref[...] = (
        jnp.dot(h_ref[...], w1_ref[...], preferred_element_type=jnp.float32)
        + jnp.dot(h_n, w2_ref[...], preferred_element_type=jnp.float32)
        + b_ref[...]
    )


def _tc_finish(h, msg, d0, d1, w1t, w2t, b2):
    R = 1000
    grid = (N_NODES // R,)
    return pl.pallas_call(
        _tc_body,
        grid=grid,
        in_specs=[
            pl.BlockSpec((R, D), lambda i: (i, 0)),
            pl.BlockSpec((1, R, D), lambda i: (0, i, 0)),
            pl.BlockSpec((1, R, D), lambda i: (1, i, 0)),
            pl.BlockSpec((R, 1), lambda i: (i, 0)),
            pl.BlockSpec((R, 1), lambda i: (i, 0)),
            pl.BlockSpec((D, D), lambda i: (0, 0)),
            pl.BlockSpec((D, D), lambda i: (0, 0)),
            pl.BlockSpec((1, D), lambda i: (0, 0)),
        ],
        out_specs=pl.BlockSpec((R, D), lambda i: (i, 0)),
        out_shape=jax.ShapeDtypeStruct((N_NODES, D), jnp.float32),
    )(h, msg, msg, d0, d1, w1t, w2t, b2)


_PERM = np.asarray(
    [32 * q + o
     for q in range(D // 32)
     for o in [2 * k for k in range(16)] + [2 * k + 1 for k in range(16)]],
    dtype=np.int32)


def kernel(h, edge_index, w, W, b):
    src2 = edge_index[0].reshape(NT, SUBS, KH)
    dst2 = edge_index[1].reshape(NT, SUBS, KH)
    w2 = w.reshape(NT, PAIRS, 2 * KH)
    h_bf = h.astype(jnp.bfloat16)
    msg, deg = _make_sc()(h_bf, src2, dst2, w2)
    w1t = W[:, :D].T
    w2t = W[:, D:].T[_PERM]
    b2 = b.reshape(1, D)
    return _tc_finish(h, msg,
                      deg[0, :N_NODES, None], deg[1, :N_NODES, None],
                      w1t, w2t, b2)
